# Initial kernel scaffold; baseline (speedup 1.0000x reference)
import jax
import jax.numpy as jnp
from jax.experimental import pallas as pl


def kernel(x, edge_index, W1, b1, W2, b2):
    raise NotImplementedError("write your pallas kernel here")


# speedup vs baseline: 17.9442x; 17.9442x over previous
import functools

import jax
import jax.numpy as jnp
from jax import lax
from jax.experimental import pallas as pl
from jax.experimental.pallas import tpu as pltpu
from jax.experimental.pallas import tpu_sc as plsc

N_NODES = 10000
D = 128
E_EDGES = 320000
NC = 2
NS = 16
K_CHUNKS = 125
C_CHUNK = 80
assert NC * NS * K_CHUNKS * C_CHUNK == E_EDGES
N_PAD = 10240
ROWS_PER_TILE = N_PAD // NS


def _sc_mesh():
    return plsc.VectorSubcoreMesh(core_axis_name="c", subcore_axis_name="s")


@functools.partial(
    pl.kernel,
    mesh=_sc_mesh(),
    out_type=jax.ShapeDtypeStruct((NC, N_PAD, 16), jnp.float32),
    scratch_types=[
        pltpu.VMEM((K_CHUNKS, C_CHUNK), jnp.int32),
        pltpu.VMEM((C_CHUNK, 16), jnp.float32),
        pltpu.VMEM((C_CHUNK, 16), jnp.float32),
        pltpu.VMEM_SHARED((N_PAD, 16), jnp.float32),
    ],
)
def _sc_degree(col_hbm, out_hbm, idx_v, ones_v, zbuf_v, acc_sh):
    c = lax.axis_index("c")
    s = lax.axis_index("s")
    base = s * ROWS_PER_TILE

    def fill_z(i, carry):
        zbuf_v[i, :] = jnp.zeros((16,), jnp.float32)
        return carry

    lax.fori_loop(0, C_CHUNK, fill_z, 0)

    def fill_o(i, carry):
        ones_v[i, :] = jnp.ones((16,), jnp.float32)
        return carry

    lax.fori_loop(0, C_CHUNK, fill_o, 0)

    for r in range(ROWS_PER_TILE // C_CHUNK):
        pltpu.sync_copy(zbuf_v, acc_sh.at[pl.ds(base + r * C_CHUNK, C_CHUNK)])
    pltpu.sync_copy(col_hbm.at[c, s], idx_v)
    plsc.subcore_barrier()

    def step(j, carry):
        pltpu.sync_copy(ones_v, acc_sh.at[idx_v.at[j]], add=True)
        return carry

    lax.fori_loop(0, K_CHUNKS, step, 0)
    plsc.subcore_barrier()
    pltpu.sync_copy(
        acc_sh.at[pl.ds(base, ROWS_PER_TILE)],
        out_hbm.at[c, pl.ds(base, ROWS_PER_TILE)],
    )


@functools.partial(
    pl.kernel,
    mesh=_sc_mesh(),
    out_type=jax.ShapeDtypeStruct((NC, N_PAD, D), jnp.float32),
    scratch_types=[
        pltpu.VMEM((K_CHUNKS, C_CHUNK), jnp.int32),
        pltpu.VMEM((K_CHUNKS, C_CHUNK), jnp.int32),
        pltpu.VMEM((C_CHUNK, D), jnp.float32),
        pltpu.VMEM_SHARED((N_PAD, D), jnp.float32),
        pltpu.SemaphoreType.DMA,
    ],
)
def _sc_aggregate(hp_hbm, row_hbm, col_hbm, out_hbm,
                  ridx_v, cidx_v, buf_v, acc_sh, sem):
    c = lax.axis_index("c")
    s = lax.axis_index("s")
    base = s * ROWS_PER_TILE

    def fill_z(t, carry):
        buf_v[t // 8, pl.ds((t % 8) * 16, 16)] = jnp.zeros((16,), jnp.float32)
        return carry

    lax.fori_loop(0, C_CHUNK * 8, fill_z, 0)

    for r in range(ROWS_PER_TILE // C_CHUNK):
        pltpu.sync_copy(buf_v, acc_sh.at[pl.ds(base + r * C_CHUNK, C_CHUNK)])
    pltpu.sync_copy(row_hbm.at[c, s], ridx_v)
    pltpu.sync_copy(col_hbm.at[c, s], cidx_v)
    plsc.subcore_barrier()

    def step(j, carry):
        pltpu.async_copy(hp_hbm.at[ridx_v.at[j]], buf_v, sem).wait()
        pltpu.sync_copy(buf_v, acc_sh.at[cidx_v.at[j]], add=True)
        return carry

    lax.fori_loop(0, K_CHUNKS, step, 0)
    plsc.subcore_barrier()
    pltpu.sync_copy(
        acc_sh.at[pl.ds(base, ROWS_PER_TILE)],
        out_hbm.at[c, pl.ds(base, ROWS_PER_TILE)],
    )


BR = 400


def _dinv_block(d0_ref, d1_ref):
    deg = d0_ref[:, 0:1] + d1_ref[:, 0:1] + 1.0
    return lax.rsqrt(deg)


def _tc_pre_body(x_ref, w_ref, d0_ref, d1_ref, hp_ref):
    dinv = _dinv_block(d0_ref, d1_ref)
    h = jnp.dot(x_ref[...], w_ref[...], preferred_element_type=jnp.float32)
    hp_ref[...] = h * dinv


def _tc_mid_body(p0_ref, p1_ref, hp_ref, b_ref, w_ref, d0_ref, d1_ref, out_ref):
    dinv = _dinv_block(d0_ref, d1_ref)
    z = dinv * (p0_ref[...] + p1_ref[...] + hp_ref[...]) + b_ref[...]
    h = jnp.maximum(z, 0.0)
    out_ref[...] = jnp.dot(h, w_ref[...], preferred_element_type=jnp.float32) * dinv


def _tc_post_body(p0_ref, p1_ref, hp_ref, b_ref, d0_ref, d1_ref, out_ref):
    dinv = _dinv_block(d0_ref, d1_ref)
    out_ref[...] = dinv * (p0_ref[...] + p1_ref[...] + hp_ref[...]) + b_ref[...]


def _row_spec():
    return pl.BlockSpec((BR, D), lambda i: (i, 0))


def _deg_spec():
    return pl.BlockSpec((BR, 16), lambda i: (i, 0))


def _full_spec():
    return pl.BlockSpec((D, D), lambda i: (0, 0))


def _bias_spec():
    return pl.BlockSpec((1, D), lambda i: (0, 0))


def _tc_pre(x, W, deg0, deg1):
    return pl.pallas_call(
        _tc_pre_body,
        grid=(N_NODES // BR,),
        in_specs=[_row_spec(), _full_spec(), _deg_spec(), _deg_spec()],
        out_specs=_row_spec(),
        out_shape=jax.ShapeDtypeStruct((N_NODES, D), jnp.float32),
    )(x, W, deg0, deg1)


def _tc_mid(p0, p1, hp, b, W, deg0, deg1):
    return pl.pallas_call(
        _tc_mid_body,
        grid=(N_NODES // BR,),
        in_specs=[_row_spec(), _row_spec(), _row_spec(), _bias_spec(),
                  _full_spec(), _deg_spec(), _deg_spec()],
        out_specs=_row_spec(),
        out_shape=jax.ShapeDtypeStruct((N_NODES, D), jnp.float32),
    )(p0, p1, hp, b, W, deg0, deg1)


def _tc_post(p0, p1, hp, b, deg0, deg1):
    return pl.pallas_call(
        _tc_post_body,
        grid=(N_NODES // BR,),
        in_specs=[_row_spec(), _row_spec(), _row_spec(), _bias_spec(),
                  _deg_spec(), _deg_spec()],
        out_specs=_row_spec(),
        out_shape=jax.ShapeDtypeStruct((N_NODES, D), jnp.float32),
    )(p0, p1, hp, b, deg0, deg1)


def kernel(x, edge_index, W1, b1, W2, b2):
    row = edge_index[0].reshape(NC, NS, K_CHUNKS, C_CHUNK)
    col = edge_index[1].reshape(NC, NS, K_CHUNKS, C_CHUNK)

    deg = _sc_degree(col)
    deg0, deg1 = deg[0, :N_NODES], deg[1, :N_NODES]

    hp1 = _tc_pre(x, W1, deg0, deg1)
    agg1 = _sc_aggregate(hp1, row, col)
    hp2 = _tc_mid(agg1[0, :N_NODES], agg1[1, :N_NODES], hp1,
                  b1.reshape(1, D), W2, deg0, deg1)
    agg2 = _sc_aggregate(hp2, row, col)
    out = _tc_post(agg2[0, :N_NODES], agg2[1, :N_NODES], hp2,
                   b2.reshape(1, D), deg0, deg1)
    return out
